# baseline (speedup 1.0000x reference)
import functools
import math

import jax
import jax.numpy as jnp
from jax import lax
from jax.experimental import pallas as pl
from jax.experimental.pallas import tpu as pltpu
from jax.experimental.pallas import tpu_sc as plsc

EMB_DIM = 64
SCALE = math.sqrt(EMB_DIM)

NUM_CORES = 2
NUM_SUBCORES = 16
NUM_WORKERS = NUM_CORES * NUM_SUBCORES
CHUNK = 128
LANES = 16
NBUF = 2


@functools.partial(jax.jit, static_argnames=("n_chunks",))
def _embed(idx, table2, n_chunks):
    n_per_w = n_chunks * CHUNK
    n_total = NUM_WORKERS * n_per_w

    mesh = plsc.VectorSubcoreMesh(
        core_axis_name="c", subcore_axis_name="s",
        num_cores=NUM_CORES, num_subcores=NUM_SUBCORES,
    )

    @functools.partial(
        pl.kernel,
        out_type=jax.ShapeDtypeStruct((n_total, EMB_DIM), jnp.float32),
        mesh=mesh,
        scratch_types=[
            pltpu.VMEM((n_chunks, CHUNK), jnp.int32),
            pltpu.VMEM((NBUF, CHUNK), jnp.int32),
            pltpu.VMEM((NBUF, CHUNK, 2 * EMB_DIM), jnp.float32),
            pltpu.VMEM((NBUF, CHUNK, EMB_DIM), jnp.float32),
            pltpu.SemaphoreType.DMA,
            pltpu.SemaphoreType.DMA,
            pltpu.SemaphoreType.DMA,
            pltpu.SemaphoreType.DMA,
        ],
        compiler_params=pltpu.CompilerParams(needs_layout_passes=False),
    )
    def body(idx_hbm, table_hbm, out_hbm, idx_v, ridx_v, g_v, o_v,
             gsem0, gsem1, osem0, osem1):
        gsems = (gsem0, gsem1)
        osems = (osem0, osem1)
        wid = lax.axis_index("s") * NUM_CORES + lax.axis_index("c")
        base = wid * n_per_w
        pltpu.sync_copy(idx_hbm.at[wid], idx_v)

        def build_and_fire(t, bb):
            def grp(g, carry):
                t16 = idx_v[t, pl.ds(g * LANES, LANES)]
                ridx_v[bb, pl.ds(g * LANES, LANES)] = (
                    lax.shift_right_logical(t16, 1))
                return carry
            lax.fori_loop(0, CHUNK // LANES, grp, 0)
            pltpu.async_copy(table_hbm.at[ridx_v.at[bb]], g_v.at[bb],
                             gsems[bb])

        for bb in range(NBUF):
            build_and_fire(bb, bb)

        iota16 = lax.iota(jnp.int32, LANES)

        def chunk_step(t, carry):
            b = lax.rem(t, NBUF)

            def per_buf(bb):
                pltpu.make_async_copy(
                    table_hbm.at[ridx_v.at[bb]], g_v.at[bb],
                    gsems[bb]).wait()

                @pl.when(t >= NBUF)
                def _():
                    pltpu.make_async_copy(
                        o_v.at[bb],
                        out_hbm.at[pl.ds(base + (t - NBUF) * CHUNK, CHUNK)],
                        osems[bb]).wait()

                def grp(g, carry2):
                    row16 = iota16 + g * LANES
                    t16 = idx_v[t, pl.ds(g * LANES, LANES)]
                    p64 = lax.shift_left(lax.bitwise_and(t16, 1), 6)
                    for k in range(EMB_DIM):
                        v = plsc.load_gather(g_v.at[bb], [row16, p64 + k])
                        plsc.store_scatter(
                            o_v.at[bb],
                            [row16, jnp.full((LANES,), k, jnp.int32)],
                            v * SCALE)
                    return carry2
                lax.fori_loop(0, CHUNK // LANES, grp, 0)

                pltpu.async_copy(
                    o_v.at[bb],
                    out_hbm.at[pl.ds(base + t * CHUNK, CHUNK)], osems[bb])

                @pl.when(t + NBUF < n_chunks)
                def _():
                    build_and_fire(t + NBUF, bb)

            for bb in range(NBUF):
                @pl.when(b == bb)
                def _(bb=bb):
                    per_buf(bb)
            return carry

        lax.fori_loop(0, n_chunks, chunk_step, 0)

        for bb in range(NBUF):
            t_last = n_chunks - NBUF + bb
            pltpu.make_async_copy(
                o_v.at[bb],
                out_hbm.at[pl.ds(base + t_last * CHUNK, CHUNK)],
                osems[bb]).wait()

    return body(idx, table2)


def kernel(tokens, embedding_weight):
    b, s = tokens.shape
    n = b * s
    assert n % (NUM_WORKERS * CHUNK) == 0
    n_chunks = n // (NUM_WORKERS * CHUNK)
    assert n_chunks % NBUF == 0
    idx = tokens.reshape(NUM_WORKERS, n_chunks, CHUNK).astype(jnp.int32)
    table2 = embedding_weight.reshape(-1, 2 * EMB_DIM)
    out = _embed(idx, table2, n_chunks)
    return out.reshape(b, s, EMB_DIM)


# speedup vs baseline: 2.7424x; 2.7424x over previous
import functools
import math

import jax
import jax.numpy as jnp
from jax import lax
from jax.experimental import pallas as pl
from jax.experimental.pallas import tpu as pltpu
from jax.experimental.pallas import tpu_sc as plsc

EMB_DIM = 64
SCALE = math.sqrt(EMB_DIM)

NUM_CORES = 2
NUM_SUBCORES = 16
NUM_WORKERS = NUM_CORES * NUM_SUBCORES
CHUNK = 128
LANES = 16
NBUF = 2


@functools.partial(jax.jit, static_argnames=("n_chunks",))
def _embed(idx, table_p, n_chunks):
    n_per_w = n_chunks * CHUNK
    n_total = NUM_WORKERS * n_per_w

    mesh = plsc.VectorSubcoreMesh(
        core_axis_name="c", subcore_axis_name="s",
        num_cores=NUM_CORES, num_subcores=NUM_SUBCORES,
    )

    @functools.partial(
        pl.kernel,
        out_type=jax.ShapeDtypeStruct((n_total, EMB_DIM), jnp.float32),
        mesh=mesh,
        scratch_types=[
            pltpu.VMEM((n_chunks, CHUNK), jnp.int32),
            pltpu.VMEM((NBUF, CHUNK, 2 * EMB_DIM), jnp.float32),
            pltpu.VMEM((NBUF, CHUNK, EMB_DIM), jnp.float32),
            pltpu.SemaphoreType.DMA,
            pltpu.SemaphoreType.DMA,
            pltpu.SemaphoreType.DMA,
            pltpu.SemaphoreType.DMA,
        ],
    )
    def body(idx_hbm, table_hbm, out_hbm, idx_v, g_v, o_v,
             gsem0, gsem1, osem0, osem1):
        gsems = (gsem0, gsem1)
        osems = (osem0, osem1)
        wid = lax.axis_index("s") * NUM_CORES + lax.axis_index("c")
        base = wid * n_per_w
        pltpu.sync_copy(idx_hbm.at[wid], idx_v)

        def fire_gather(t, bb):
            pltpu.async_copy(table_hbm.at[idx_v.at[t]], g_v.at[bb],
                             gsems[bb])

        for bb in range(NBUF):
            fire_gather(bb, bb)

        def chunk_step(t, carry):
            b = lax.rem(t, NBUF)

            def per_buf(bb):
                pltpu.make_async_copy(
                    table_hbm.at[idx_v.at[t]], g_v.at[bb], gsems[bb]).wait()

                @pl.when(t >= NBUF)
                def _():
                    pltpu.make_async_copy(
                        o_v.at[bb],
                        out_hbm.at[pl.ds(base + (t - NBUF) * CHUNK, CHUNK)],
                        osems[bb]).wait()

                def row(i, carry2):
                    for j in range(EMB_DIM // LANES):
                        sl = pl.ds(j * LANES, LANES)
                        o_v[bb, i, sl] = g_v[bb, i, sl] * SCALE
                    return carry2
                lax.fori_loop(0, CHUNK, row, 0)

                pltpu.async_copy(
                    o_v.at[bb],
                    out_hbm.at[pl.ds(base + t * CHUNK, CHUNK)], osems[bb])

                @pl.when(t + NBUF < n_chunks)
                def _():
                    fire_gather(t + NBUF, bb)

            for bb in range(NBUF):
                @pl.when(b == bb)
                def _(bb=bb):
                    per_buf(bb)
            return carry

        lax.fori_loop(0, n_chunks, chunk_step, 0)

        for bb in range(NBUF):
            t_last = n_chunks - NBUF + bb
            pltpu.make_async_copy(
                o_v.at[bb],
                out_hbm.at[pl.ds(base + t_last * CHUNK, CHUNK)],
                osems[bb]).wait()

    return body(idx, table_p)


def kernel(tokens, embedding_weight):
    b, s = tokens.shape
    n = b * s
    assert n % (NUM_WORKERS * CHUNK) == 0
    n_chunks = n // (NUM_WORKERS * CHUNK)
    assert n_chunks % NBUF == 0
    idx = tokens.reshape(NUM_WORKERS, n_chunks, CHUNK).astype(jnp.int32)
    table_p = jnp.pad(embedding_weight, ((0, 0), (0, EMB_DIM)))
    out = _embed(idx, table_p, n_chunks)
    return out.reshape(b, s, EMB_DIM)


# speedup vs baseline: 3.3548x; 1.2233x over previous
import functools
import math

import jax
import jax.numpy as jnp
from jax import lax
from jax.experimental import pallas as pl
from jax.experimental.pallas import tpu as pltpu
from jax.experimental.pallas import tpu_sc as plsc

EMB_DIM = 64
SCALE = math.sqrt(EMB_DIM)

NUM_CORES = 2
NUM_SUBCORES = 16
NUM_WORKERS = NUM_CORES * NUM_SUBCORES
CHUNK = 128
LANES = 16
NBUF = 2


@functools.partial(jax.jit, static_argnames=("n_chunks",))
def _embed(idx, table_p, n_chunks):
    n_per_w = n_chunks * CHUNK
    n_total = NUM_WORKERS * n_per_w

    mesh = plsc.VectorSubcoreMesh(
        core_axis_name="c", subcore_axis_name="s",
        num_cores=NUM_CORES, num_subcores=NUM_SUBCORES,
    )

    @functools.partial(
        pl.kernel,
        out_type=jax.ShapeDtypeStruct((n_total, EMB_DIM), jnp.float32),
        mesh=mesh,
        scratch_types=[
            pltpu.VMEM((n_chunks, CHUNK), jnp.int32),
            pltpu.VMEM((NBUF, CHUNK, EMB_DIM), jnp.float32),
            pltpu.VMEM((NBUF, CHUNK, EMB_DIM), jnp.float32),
            pltpu.SemaphoreType.DMA,
            pltpu.SemaphoreType.DMA,
            pltpu.SemaphoreType.DMA,
            pltpu.SemaphoreType.DMA,
        ],
    )
    def body(idx_hbm, table_hbm, out_hbm, idx_v, g_v, o_v,
             gsem0, gsem1, osem0, osem1):
        gsems = (gsem0, gsem1)
        osems = (osem0, osem1)
        wid = lax.axis_index("s") * NUM_CORES + lax.axis_index("c")
        base = wid * n_per_w
        pltpu.sync_copy(idx_hbm.at[wid], idx_v)

        def fire_gather(t, bb):
            def grp_dma(g, carry):
                t16 = idx_v[t, pl.ds(g * LANES, LANES)]
                for i in range(LANES):
                    pltpu.async_copy(
                        table_hbm.at[t16[i]],
                        g_v.at[bb, g * LANES + i], gsems[bb])
                return carry
            lax.fori_loop(0, CHUNK // LANES, grp_dma, 0)

        for bb in range(NBUF):
            fire_gather(bb, bb)

        def chunk_step(t, carry):
            b = lax.rem(t, NBUF)

            def per_buf(bb):
                pltpu.make_async_copy(
                    table_hbm.at[pl.ds(0, CHUNK)], g_v.at[bb],
                    gsems[bb]).wait()

                @pl.when(t >= NBUF)
                def _():
                    pltpu.make_async_copy(
                        o_v.at[bb],
                        out_hbm.at[pl.ds(base + (t - NBUF) * CHUNK, CHUNK)],
                        osems[bb]).wait()

                def row(i, carry2):
                    for j in range(EMB_DIM // LANES):
                        sl = pl.ds(j * LANES, LANES)
                        o_v[bb, i, sl] = g_v[bb, i, sl] * SCALE
                    return carry2
                lax.fori_loop(0, CHUNK, row, 0)

                pltpu.async_copy(
                    o_v.at[bb],
                    out_hbm.at[pl.ds(base + t * CHUNK, CHUNK)], osems[bb])

                @pl.when(t + NBUF < n_chunks)
                def _():
                    fire_gather(t + NBUF, bb)

            for bb in range(NBUF):
                @pl.when(b == bb)
                def _(bb=bb):
                    per_buf(bb)
            return carry

        lax.fori_loop(0, n_chunks, chunk_step, 0)

        for bb in range(NBUF):
            t_last = n_chunks - NBUF + bb
            pltpu.make_async_copy(
                o_v.at[bb],
                out_hbm.at[pl.ds(base + t_last * CHUNK, CHUNK)],
                osems[bb]).wait()

    return body(idx, table_p)


def kernel(tokens, embedding_weight):
    b, s = tokens.shape
    n = b * s
    assert n % (NUM_WORKERS * CHUNK) == 0
    n_chunks = n // (NUM_WORKERS * CHUNK)
    assert n_chunks % NBUF == 0
    idx = tokens.reshape(NUM_WORKERS, n_chunks, CHUNK).astype(jnp.int32)
    out = _embed(idx, embedding_weight, n_chunks)
    return out.reshape(b, s, EMB_DIM)
